# Initial kernel scaffold; baseline (speedup 1.0000x reference)
import jax
import jax.numpy as jnp
from jax.experimental import pallas as pl


def kernel(x, weight):
    raise NotImplementedError("write your pallas kernel here")


# speedup vs baseline: 1.4594x; 1.4594x over previous
import jax
import jax.numpy as jnp
from jax import lax
from jax.experimental import pallas as pl
from jax.experimental.pallas import tpu as pltpu
from jax.experimental.pallas import tpu_sc as plsc

NC = 2
NS = 16
NW = NC * NS

EMB = 32
TOTAL = 4096 * 200
IDX_MINOR = 128
IDX_ROWS = TOTAL // IDX_MINOR
IDX_ROWS_PER_W = IDX_ROWS // NW
G = 8
NGROUPS = IDX_ROWS_PER_W // G
GROUP_ROWS = G * IDX_MINOR


def _emb_lookup(idx, weight):
    mesh = plsc.VectorSubcoreMesh(
        core_axis_name="c", subcore_axis_name="s", num_cores=NC, num_subcores=NS
    )

    def body(idx_hbm, table_hbm, out_hbm, idx_v, rows_v, sem):
        wid = lax.axis_index("s") * NC + lax.axis_index("c")

        @pl.loop(0, NGROUPS)
        def _group(g):
            irow0 = wid * IDX_ROWS_PER_W + g * G
            pltpu.sync_copy(idx_hbm.at[pl.ds(irow0, G)], idx_v)
            copies = [
                pltpu.async_copy(
                    table_hbm.at[idx_v.at[j]],
                    rows_v.at[pl.ds(j * IDX_MINOR, IDX_MINOR)],
                    sem,
                )
                for j in range(G)
            ]
            for c in copies:
                c.wait()
            pltpu.sync_copy(rows_v, out_hbm.at[pl.ds(irow0 * IDX_MINOR, GROUP_ROWS)])

    run = pl.kernel(
        body,
        out_type=jax.ShapeDtypeStruct((TOTAL, EMB), jnp.float32),
        mesh=mesh,
        scratch_types=[
            pltpu.VMEM((G, IDX_MINOR), jnp.int32),
            pltpu.VMEM((GROUP_ROWS, EMB), jnp.float32),
            pltpu.SemaphoreType.DMA,
        ],
        compiler_params=pltpu.CompilerParams(use_tc_tiling_on_sc=False),
    )
    return run(idx, weight)


def kernel(x, weight):
    idx = x.reshape(-1).astype(jnp.int32).reshape(IDX_ROWS, IDX_MINOR)
    out = _emb_lookup(idx, weight)
    return out.reshape(x.shape + (weight.shape[-1],))


# speedup vs baseline: 1.4988x; 1.0270x over previous
import jax
import jax.numpy as jnp
from jax import lax
from jax.experimental import pallas as pl
from jax.experimental.pallas import tpu as pltpu
from jax.experimental.pallas import tpu_sc as plsc

NC = 2
NS = 16
NW = NC * NS

EMB = 32
TOTAL = 4096 * 200
IDX_MINOR = 128
IDX_ROWS = TOTAL // IDX_MINOR
IDX_ROWS_PER_W = IDX_ROWS // NW
NB = 2
G = 10
NGROUPS = IDX_ROWS_PER_W // G
GROUP_ROWS = G * IDX_MINOR


def _emb_lookup(idx, weight):
    mesh = plsc.VectorSubcoreMesh(
        core_axis_name="c", subcore_axis_name="s", num_cores=NC, num_subcores=NS
    )

    def body(idx_hbm, table_hbm, out_hbm, idx_v, rows_v, gsems, osems):
        wid = lax.axis_index("s") * NC + lax.axis_index("c")

        @pl.loop(0, NGROUPS, step=NB)
        def _outer(g0):
            for b in range(NB):
                @pl.when(g0 > 0)
                def _wait_prev_store():
                    pltpu.make_async_copy(
                        out_hbm.at[pl.ds(0, GROUP_ROWS)], rows_v.at[b], osems[b]
                    ).wait()

                irow0 = wid * IDX_ROWS_PER_W + (g0 + b) * G
                pltpu.sync_copy(idx_hbm.at[pl.ds(irow0, G)], idx_v.at[b])
                for j in range(G):
                    pltpu.async_copy(
                        table_hbm.at[idx_v.at[b].at[j]],
                        rows_v.at[b].at[pl.ds(j * IDX_MINOR, IDX_MINOR)],
                        gsems[b],
                    )
            for b in range(NB):
                pltpu.make_async_copy(
                    out_hbm.at[pl.ds(0, GROUP_ROWS)], rows_v.at[b], gsems[b]
                ).wait()
                orow0 = (wid * IDX_ROWS_PER_W + (g0 + b) * G) * IDX_MINOR
                pltpu.async_copy(
                    rows_v.at[b], out_hbm.at[pl.ds(orow0, GROUP_ROWS)], osems[b]
                )

        for b in range(NB):
            pltpu.make_async_copy(
                out_hbm.at[pl.ds(0, GROUP_ROWS)], rows_v.at[b], osems[b]
            ).wait()

    run = pl.kernel(
        body,
        out_type=jax.ShapeDtypeStruct((TOTAL, EMB), jnp.float32),
        mesh=mesh,
        scratch_types=[
            pltpu.VMEM((NB, G, IDX_MINOR), jnp.int32),
            pltpu.VMEM((NB, GROUP_ROWS, EMB), jnp.float32),
            [pltpu.SemaphoreType.DMA] * NB,
            [pltpu.SemaphoreType.DMA] * NB,
        ],
        compiler_params=pltpu.CompilerParams(use_tc_tiling_on_sc=False),
    )
    return run(idx, weight)


def kernel(x, weight):
    idx = x.reshape(-1).astype(jnp.int32).reshape(IDX_ROWS, IDX_MINOR)
    out = _emb_lookup(idx, weight)
    return out.reshape(x.shape + (weight.shape[-1],))
